# baseline (speedup 1.0000x reference)
import functools

import jax
import jax.numpy as jnp
from jax import lax
from jax.experimental import pallas as pl
from jax.experimental.pallas import tpu as pltpu
from jax.experimental.pallas import tpu_sc as plsc

_D = 64
_NC, _NS = 2, 16
_NW = _NC * _NS
_CHUNK = 512
_NBUF = 3


@functools.cache
def _make_gather(n):
    b_per_w = n // _NW
    n_chunks = b_per_w // _CHUNK
    n_steady = ((n_chunks - _NBUF) // _NBUF) * _NBUF
    mesh = plsc.VectorSubcoreMesh(
        core_axis_name="c", subcore_axis_name="s",
        num_cores=_NC, num_subcores=_NS,
    )

    @functools.partial(
        pl.kernel,
        out_type=jax.ShapeDtypeStruct((n, 2 * _D), jnp.float32),
        mesh=mesh,
        scratch_types=[
            pltpu.VMEM((b_per_w,), jnp.int32),
            pltpu.VMEM((_NBUF, _CHUNK, _D), jnp.float32),
            pltpu.SemaphoreType.DMA,
            pltpu.SemaphoreType.DMA((_NBUF,)),
            pltpu.SemaphoreType.DMA((_NBUF,)),
        ],
        compiler_params=pltpu.CompilerParams(use_tc_tiling_on_sc=False),
    )
    def k(idx_hbm, table_hbm, out_hbm, idx_v, rows_v, sem_i, sem_g, sem_o):
        wid = lax.axis_index("s") * _NC + lax.axis_index("c")
        base = wid * b_per_w

        pltpu.async_copy(
            idx_hbm.at[pl.ds(base, b_per_w)], idx_v, sem_i
        ).wait()

        def start_gather(g, b):
            pltpu.async_copy(
                table_hbm.at[idx_v.at[pl.ds(g * _CHUNK, _CHUNK)]],
                rows_v.at[b],
                sem_g.at[b],
            )

        def wait_gather(b):
            pltpu.make_async_copy(
                table_hbm.at[pl.ds(0, _CHUNK)], rows_v.at[b], sem_g.at[b]
            ).wait()

        def wait_write(b):
            pltpu.make_async_copy(
                rows_v.at[b],
                out_hbm.at[pl.ds(base, _CHUNK), pl.ds(0, _D)],
                sem_o.at[b],
            ).wait()

        def start_write(g, b):
            pltpu.async_copy(
                rows_v.at[b],
                out_hbm.at[pl.ds(base + g * _CHUNK, _CHUNK), pl.ds(0, _D)],
                sem_o.at[b],
            )

        for b in range(_NBUF):
            start_gather(b, b)

        def body(i, carry):
            g0 = i * _NBUF
            for b in range(_NBUF):
                g = g0 + b
                wait_gather(b)
                start_write(g, b)
                wait_write(b)
                start_gather(g + _NBUF, b)
            return carry

        lax.fori_loop(0, n_steady // _NBUF, body, 0)

        for g in range(n_steady, n_chunks):
            b = g % _NBUF
            wait_gather(b)
            start_write(g, b)
            wait_write(b)
            if g + _NBUF < n_chunks:
                start_gather(g + _NBUF, b)

    return k


def kernel(x, table):
    b, h = x.shape
    flat = x.reshape(b * h).astype(jnp.int32)
    out = _make_gather(b * h)(flat, table)
    return out[:, :_D].reshape(b, h, _D)


# speedup vs baseline: 1.0010x; 1.0010x over previous
import functools

import jax
import jax.numpy as jnp
from jax import lax
from jax.experimental import pallas as pl
from jax.experimental.pallas import tpu as pltpu
from jax.experimental.pallas import tpu_sc as plsc

_D = 64
_NC, _NS = 2, 16
_NW = _NC * _NS
_CHUNK = 640
_NBUF = 2


@functools.cache
def _make_gather(n):
    b_per_w = n // _NW
    n_chunks = b_per_w // _CHUNK
    n_steady = ((n_chunks - _NBUF) // _NBUF) * _NBUF
    mesh = plsc.VectorSubcoreMesh(
        core_axis_name="c", subcore_axis_name="s",
        num_cores=_NC, num_subcores=_NS,
    )

    @functools.partial(
        pl.kernel,
        out_type=jax.ShapeDtypeStruct((n, 2 * _D), jnp.float32),
        mesh=mesh,
        scratch_types=[
            pltpu.VMEM((b_per_w,), jnp.int32),
            pltpu.VMEM((_NBUF, _CHUNK, _D), jnp.float32),
            pltpu.SemaphoreType.DMA,
            pltpu.SemaphoreType.DMA((_NBUF,)),
            pltpu.SemaphoreType.DMA((_NBUF,)),
        ],
        compiler_params=pltpu.CompilerParams(use_tc_tiling_on_sc=False),
    )
    def k(idx_hbm, table_hbm, out_hbm, idx_v, rows_v, sem_i, sem_g, sem_o):
        wid = lax.axis_index("s") * _NC + lax.axis_index("c")
        base = wid * b_per_w

        pltpu.async_copy(
            idx_hbm.at[pl.ds(base, b_per_w)], idx_v, sem_i
        ).wait()

        def start_gather(g, b):
            pltpu.async_copy(
                table_hbm.at[idx_v.at[pl.ds(g * _CHUNK, _CHUNK)]],
                rows_v.at[b],
                sem_g.at[b],
            )

        def wait_gather(b):
            pltpu.make_async_copy(
                table_hbm.at[pl.ds(0, _CHUNK)], rows_v.at[b], sem_g.at[b]
            ).wait()

        def wait_write(b):
            pltpu.make_async_copy(
                rows_v.at[b],
                out_hbm.at[pl.ds(base, _CHUNK), pl.ds(0, _D)],
                sem_o.at[b],
            ).wait()

        def start_write(g, b):
            pltpu.async_copy(
                rows_v.at[b],
                out_hbm.at[pl.ds(base + g * _CHUNK, _CHUNK), pl.ds(0, _D)],
                sem_o.at[b],
            )

        for b in range(_NBUF):
            start_gather(b, b)

        def body(i, carry):
            g0 = i * _NBUF
            for b in range(_NBUF):
                g = g0 + b
                wait_gather(b)
                start_write(g, b)
                wait_write(b)
                start_gather(g + _NBUF, b)
            return carry

        lax.fori_loop(0, n_steady // _NBUF, body, 0)

        for g in range(n_steady, n_chunks):
            b = g % _NBUF
            wait_gather(b)
            start_write(g, b)
            wait_write(b)
            if g + _NBUF < n_chunks:
                start_gather(g + _NBUF, b)

    return k


def kernel(x, table):
    b, h = x.shape
    flat = x.reshape(b * h).astype(jnp.int32)
    out = _make_gather(b * h)(flat, table)
    return out[:, :_D].reshape(b, h, _D)
